# initial kernel scaffold (unmeasured)
import jax
import jax.numpy as jnp
from jax import lax
from jax.experimental import pallas as pl
from jax.experimental.pallas import tpu as pltpu

N_DEV = 32
B = 4096
BB = B // N_DEV
D = 128


def kernel(x, Win0, Wout0, Win1, Wout1, Win2, Wout2):
    def body(x_ref, win0_ref, wout0_ref, win1_ref, wout1_ref, win2_ref,
             wout2_ref, out_ref,
             xfull, partial, rs_buf, stage_bf, stage_f32,
             ag_send, ag_recv, rs_send, rs_recv):
        me = lax.axis_index("i")
        row_me = pl.ds(me * BB, BB)

        def broadcast(src_ref, dst_ref, send_sems, recv_sems):
            rdmas = []
            for k in range(1, N_DEV):
                dst = (me + k) % N_DEV
                r = pltpu.make_async_remote_copy(
                    src_ref=src_ref,
                    dst_ref=dst_ref,
                    send_sem=send_sems.at[k],
                    recv_sem=recv_sems.at[k],
                    device_id=(dst,),
                    device_id_type=pl.DeviceIdType.MESH,
                )
                r.start()
                rdmas.append(r)
            for r in rdmas:
                r.wait()

        stage_bf[...] = x_ref[...].astype(jnp.bfloat16)
        xfull[row_me, :] = stage_bf[...]
        broadcast(stage_bf, xfull.at[row_me], ag_send, ag_recv)

        X = xfull[...]

        layers = [(win0_ref, wout0_ref), (win1_ref, wout1_ref),
                  (win2_ref, wout2_ref)]
        for l, (win_ref, wout_ref) in enumerate(layers):
            W1 = win_ref[...].astype(jnp.bfloat16)
            W2 = wout_ref[...].astype(jnp.bfloat16)
            h = jnp.dot(X, W1, preferred_element_type=jnp.float32)
            h = jnp.maximum(h, 0.0).astype(jnp.bfloat16)
            p = jnp.dot(h, W2, preferred_element_type=jnp.float32)
            partial[...] = p

            rdmas = []
            for k in range(1, N_DEV):
                dst = (me + k) % N_DEV
                r = pltpu.make_async_remote_copy(
                    src_ref=partial.at[pl.ds(dst * BB, BB)],
                    dst_ref=rs_buf.at[k],
                    send_sem=rs_send.at[k],
                    recv_sem=rs_recv.at[k],
                    device_id=(dst,),
                    device_id_type=pl.DeviceIdType.MESH,
                )
                r.start()
                rdmas.append(r)
            for r in rdmas:
                r.wait()

            red = partial[row_me]
            red = red + jnp.sum(rs_buf[1:, :, :], axis=0)

            if l < 2:
                stage_bf[...] = red.astype(jnp.bfloat16)
                xfull[row_me, :] = stage_bf[...]
                broadcast(stage_bf, xfull.at[row_me], ag_send, ag_recv)
                X = xfull[...]
            else:
                stage_f32[...] = red
                out_ref[row_me, :] = red
                broadcast(stage_f32, out_ref.at[row_me], ag_send, ag_recv)

    return pl.pallas_call(
        body,
        out_shape=jax.ShapeDtypeStruct((B, D), jnp.float32),
        in_specs=[pl.BlockSpec(memory_space=pltpu.VMEM)] * 7,
        out_specs=pl.BlockSpec(memory_space=pltpu.VMEM),
        scratch_shapes=[
            pltpu.VMEM((B, D), jnp.bfloat16),
            pltpu.VMEM((B, D), jnp.float32),
            pltpu.VMEM((N_DEV, BB, D), jnp.float32),
            pltpu.VMEM((BB, D), jnp.bfloat16),
            pltpu.VMEM((BB, D), jnp.float32),
            pltpu.SemaphoreType.DMA((N_DEV,)),
            pltpu.SemaphoreType.DMA((N_DEV,)),
            pltpu.SemaphoreType.DMA((N_DEV,)),
            pltpu.SemaphoreType.DMA((N_DEV,)),
        ],
        compiler_params=pltpu.CompilerParams(collective_id=0),
    )(x, Win0, Wout0, Win1, Wout1, Win2, Wout2)


# baseline (device time: 172344 ns/iter reference)
import jax
import jax.numpy as jnp
from jax import lax
from jax.experimental import pallas as pl
from jax.experimental.pallas import tpu as pltpu

N_DEV = 32
B = 4096
BB = B // N_DEV
D = 128


def kernel(x, Win0, Wout0, Win1, Wout1, Win2, Wout2):
    def body(x_ref, win0_ref, wout0_ref, win1_ref, wout1_ref, win2_ref,
             wout2_ref, out_ref,
             xfull, partial, rs_buf, stage_bf, stage_f32,
             ag_send, ag_recv, rs_send, rs_recv):
        me = lax.axis_index("i")
        row_me = pl.ds(me * BB, BB)

        def broadcast(src_ref, dst_ref, send_sems, recv_sems):
            rdmas = []
            for k in range(1, N_DEV):
                dst = (me + k) % N_DEV
                r = pltpu.make_async_remote_copy(
                    src_ref=src_ref,
                    dst_ref=dst_ref,
                    send_sem=send_sems.at[k],
                    recv_sem=recv_sems.at[k],
                    device_id=(dst,),
                    device_id_type=pl.DeviceIdType.MESH,
                )
                r.start()
                rdmas.append(r)
            for r in rdmas:
                r.wait()

        stage_bf[...] = x_ref[...].astype(jnp.bfloat16)
        xfull[row_me, :] = stage_bf[...]
        broadcast(stage_bf, xfull.at[row_me], ag_send, ag_recv)

        X = xfull[...]

        layers = [(win0_ref, wout0_ref), (win1_ref, wout1_ref),
                  (win2_ref, wout2_ref)]
        for l, (win_ref, wout_ref) in enumerate(layers):
            W1 = win_ref[...].astype(jnp.bfloat16)
            W2 = wout_ref[...].astype(jnp.bfloat16)
            h = jnp.dot(X, W1, preferred_element_type=jnp.float32)
            h = jnp.maximum(h, 0.0).astype(jnp.bfloat16)
            p = jnp.dot(h, W2, preferred_element_type=jnp.float32)
            partial[...] = p

            rdmas = []
            for k in range(1, N_DEV):
                dst = (me + k) % N_DEV
                r = pltpu.make_async_remote_copy(
                    src_ref=partial.at[pl.ds(dst * BB, BB)],
                    dst_ref=rs_buf.at[k],
                    send_sem=rs_send.at[k],
                    recv_sem=rs_recv.at[k],
                    device_id=(dst,),
                    device_id_type=pl.DeviceIdType.MESH,
                )
                r.start()
                rdmas.append(r)
            for r in rdmas:
                r.wait()

            red = partial[row_me]
            red = red + jnp.sum(rs_buf[1:, :, :], axis=0)

            if l < 2:
                stage_bf[...] = red.astype(jnp.bfloat16)
                xfull[row_me, :] = stage_bf[...]
                broadcast(stage_bf, xfull.at[row_me], ag_send, ag_recv)
                X = xfull[...]
            else:
                stage_f32[...] = red
                out_ref[row_me, :] = red
                broadcast(stage_f32, out_ref.at[row_me], ag_send, ag_recv)

    return pl.pallas_call(
        body,
        out_shape=jax.ShapeDtypeStruct((B, D), jnp.float32),
        in_specs=[pl.BlockSpec(memory_space=pltpu.VMEM)] * 7,
        out_specs=pl.BlockSpec(memory_space=pltpu.VMEM),
        scratch_shapes=[
            pltpu.VMEM((B, D), jnp.bfloat16),
            pltpu.VMEM((B, D), jnp.float32),
            pltpu.VMEM((N_DEV, BB, D), jnp.float32),
            pltpu.VMEM((BB, D), jnp.bfloat16),
            pltpu.VMEM((BB, D), jnp.float32),
            pltpu.SemaphoreType.DMA((N_DEV,)),
            pltpu.SemaphoreType.DMA((N_DEV,)),
            pltpu.SemaphoreType.DMA((N_DEV,)),
            pltpu.SemaphoreType.DMA((N_DEV,)),
        ],
    )(x, Win0, Wout0, Win1, Wout1, Win2, Wout2)


# device time: 121902 ns/iter; 1.4138x vs baseline; 1.4138x over previous
import jax
import jax.numpy as jnp
from jax import lax
from jax.experimental import pallas as pl
from jax.experimental.pallas import tpu as pltpu

N_DEV = 32
B = 4096
BB = B // N_DEV
D = 128


def kernel(x, Win0, Wout0, Win1, Wout1, Win2, Wout2):
    def body(x_ref, win0_ref, wout0_ref, win1_ref, wout1_ref, win2_ref,
             wout2_ref, out_ref,
             xfull, partial, rs_buf, stage_bf,
             ag_send, ag_recv, rs_send, rs_recv):
        me = lax.axis_index("i")
        row_me = pl.ds(me * BB, BB)

        def broadcast(src_ref, dst_ref, send_sems, recv_sems):
            rdmas = []
            for k in range(1, N_DEV):
                dst = (me + k) % N_DEV
                r = pltpu.make_async_remote_copy(
                    src_ref=src_ref,
                    dst_ref=dst_ref,
                    send_sem=send_sems.at[k],
                    recv_sem=recv_sems.at[k],
                    device_id=(dst,),
                    device_id_type=pl.DeviceIdType.MESH,
                )
                r.start()
                rdmas.append(r)
            for r in rdmas:
                r.wait()

        stage_bf[...] = x_ref[...].astype(jnp.bfloat16)
        xfull[row_me, :] = stage_bf[...]
        broadcast(stage_bf, xfull.at[row_me], ag_send, ag_recv)

        X = xfull[...]

        layers = [(win0_ref, wout0_ref), (win1_ref, wout1_ref),
                  (win2_ref, wout2_ref)]
        for l, (win_ref, wout_ref) in enumerate(layers):
            W1 = win_ref[...].astype(jnp.bfloat16)
            W2 = wout_ref[...].astype(jnp.bfloat16)
            h = jnp.dot(X, W1, preferred_element_type=jnp.float32)
            h = jnp.maximum(h, 0.0).astype(jnp.bfloat16)
            p = jnp.dot(h, W2, preferred_element_type=jnp.float32)
            partial[...] = p.astype(jnp.bfloat16)

            rdmas = []
            for k in range(1, N_DEV):
                dst = (me + k) % N_DEV
                r = pltpu.make_async_remote_copy(
                    src_ref=partial.at[pl.ds(dst * BB, BB)],
                    dst_ref=rs_buf.at[k],
                    send_sem=rs_send.at[k],
                    recv_sem=rs_recv.at[k],
                    device_id=(dst,),
                    device_id_type=pl.DeviceIdType.MESH,
                )
                r.start()
                rdmas.append(r)
            for r in rdmas:
                r.wait()

            red = partial[row_me].astype(jnp.float32)
            red = red + jnp.sum(rs_buf[1:, :, :].astype(jnp.float32), axis=0)

            stage_bf[...] = red.astype(jnp.bfloat16)
            xfull[row_me, :] = stage_bf[...]
            broadcast(stage_bf, xfull.at[row_me], ag_send, ag_recv)
            if l < 2:
                X = xfull[...]
            else:
                out_ref[...] = xfull[...].astype(jnp.float32)

    return pl.pallas_call(
        body,
        out_shape=jax.ShapeDtypeStruct((B, D), jnp.float32),
        in_specs=[pl.BlockSpec(memory_space=pltpu.VMEM)] * 7,
        out_specs=pl.BlockSpec(memory_space=pltpu.VMEM),
        scratch_shapes=[
            pltpu.VMEM((B, D), jnp.bfloat16),
            pltpu.VMEM((B, D), jnp.bfloat16),
            pltpu.VMEM((N_DEV, BB, D), jnp.bfloat16),
            pltpu.VMEM((BB, D), jnp.bfloat16),
            pltpu.SemaphoreType.DMA((N_DEV,)),
            pltpu.SemaphoreType.DMA((N_DEV,)),
            pltpu.SemaphoreType.DMA((N_DEV,)),
            pltpu.SemaphoreType.DMA((N_DEV,)),
        ],
    )(x, Win0, Wout0, Win1, Wout1, Win2, Wout2)
